# Initial kernel scaffold; baseline (speedup 1.0000x reference)
import functools

import jax
import jax.numpy as jnp
import numpy as np
from jax import lax
from jax.experimental import pallas as pl
from jax.experimental.pallas import tpu as pltpu
from jax.experimental.pallas import tpu_sc as plsc

N_NODES = 10000
N_EDGES = N_NODES * 16
NUM_RBF = 16
NUM_POS = 16
EPS = 1e-8

IBLK = 128
EP = 163840
NBLK = EP // IBLK
EB = 1024
NB = 1000
TRASH = N_NODES
NP_ACC = N_NODES + 16

_SC_INFO = plsc.get_sparse_core_info()
_NC = _SC_INFO.num_cores
_NS = _SC_INFO.num_subcores
_NW = _NC * _NS


def _sc_gather(table, idx2d):
    nblk = idx2d.shape[0]
    d = table.shape[1]
    per_w = nblk // _NW
    mesh = plsc.VectorSubcoreMesh(core_axis_name="c", subcore_axis_name="s")

    @functools.partial(
        pl.kernel, mesh=mesh,
        out_type=jax.ShapeDtypeStruct((nblk * IBLK, d), jnp.float32),
        scratch_types=[
            pltpu.VMEM((IBLK,), jnp.int32),
            pltpu.VMEM((IBLK, d), jnp.float32),
            pltpu.SemaphoreType.DMA,
        ],
    )
    def k(table_hbm, idx_hbm, out_hbm, idx_v, rows_v, sem):
        wid = lax.axis_index("s") * _NC + lax.axis_index("c")

        def body(c, carry):
            b = wid * per_w + c
            pltpu.sync_copy(idx_hbm.at[b], idx_v)
            pltpu.async_copy(table_hbm.at[idx_v], rows_v, sem).wait()
            pltpu.sync_copy(rows_v, out_hbm.at[pl.ds(b * IBLK, IBLK)])
            return carry

        lax.fori_loop(0, per_w, body, 0)

    return k(table, idx2d)


def _sc_scatter_add(vals, idx2d):
    d = vals.shape[1]
    per_w = NBLK // _NW
    zrows = NP_ACC // _NS
    orows = N_NODES // _NS
    mesh = plsc.VectorSubcoreMesh(core_axis_name="c", subcore_axis_name="s")

    @functools.partial(
        pl.kernel, mesh=mesh,
        out_type=jax.ShapeDtypeStruct((2 * N_NODES, d), jnp.float32),
        scratch_types=[
            pltpu.VMEM((IBLK,), jnp.int32),
            pltpu.VMEM((IBLK, d), jnp.float32),
            pltpu.VMEM((zrows, d), jnp.float32),
            pltpu.VMEM_SHARED((NP_ACC, d), jnp.float32),
        ],
    )
    def k(vals_hbm, idx_hbm, out_hbm, idx_v, vbuf, zbuf, acc):
        cid = lax.axis_index("c")
        sid = lax.axis_index("s")
        wid = sid * _NC + cid

        def zrow(r, carry):
            for q in range(d // 16):
                zbuf[r, pl.ds(q * 16, 16)] = jnp.zeros((16,), jnp.float32)
            return carry

        lax.fori_loop(0, zrows, zrow, 0)
        pltpu.sync_copy(zbuf, acc.at[pl.ds(sid * zrows, zrows)])
        plsc.subcore_barrier()

        def body(c, carry):
            b = wid * per_w + c
            pltpu.sync_copy(idx_hbm.at[b], idx_v)
            pltpu.sync_copy(vals_hbm.at[pl.ds(b * IBLK, IBLK)], vbuf)
            pltpu.sync_copy(vbuf, acc.at[idx_v], add=True)
            return carry

        lax.fori_loop(0, per_w, body, 0)
        plsc.subcore_barrier()

        pltpu.sync_copy(acc.at[pl.ds(sid * orows, orows)],
                        zbuf.at[pl.ds(0, orows)])
        pltpu.sync_copy(zbuf.at[pl.ds(0, orows)],
                        out_hbm.at[pl.ds(cid * N_NODES + sid * orows, orows)])

    return k(vals, idx2d)


def _mm(a, b):
    return jnp.dot(a, b, preferred_element_type=jnp.float32)


def _rot_in_blk(rc, v_c):
    def r(a, b):
        return rc[:, 3 + 3 * a + b:4 + 3 * a + b]
    return [r(0, i) * v_c[0] + r(1, i) * v_c[1] + r(2, i) * v_c[2]
            for i in range(3)]


def _rot_out_blk(rc, v_c):
    def r(a, b):
        return rc[:, 3 + 3 * a + b:4 + 3 * a + b]
    return [r(i, 0) * v_c[0] + r(i, 1) * v_c[1] + r(i, 2) * v_c[2]
            for i in range(3)]


def _ln_blk(s, v_c, nvec):
    mu = jnp.mean(s, axis=-1, keepdims=True)
    var = jnp.mean((s - mu) * (s - mu), axis=-1, keepdims=True)
    s = (s - mu) * lax.rsqrt(var + 1e-5)
    sq = v_c[0] * v_c[0] + v_c[1] * v_c[1] + v_c[2] * v_c[2]
    rms = lax.rsqrt(jnp.sum(sq, axis=-1, keepdims=True) / nvec + EPS)
    return s, [vc * rms for vc in v_c]


def _vnorm(vh_c):
    return jnp.sqrt(vh_c[0] * vh_c[0] + vh_c[1] * vh_c[1]
                    + vh_c[2] * vh_c[2] + EPS)


def _nan_to_num(x):
    x = jnp.where(x != x, 0.0, x)
    return jnp.clip(x, -3.4028235e38, 3.4028235e38)


def _vcols(h, base, nv):
    return [h[:, base + c * nv: base + (c + 1) * nv] for c in range(3)]


def _row_spec(b, d):
    return pl.BlockSpec((b, d), lambda g: (g, 0))


def _w_spec(shape):
    nd = len(shape)
    return pl.BlockSpec(shape, lambda g: (0,) * nd)


def _node_init(node_s, node_v9, pos_CA, pos_C, pos_N, w1, w2):
    def body(ns, nv, ca, cc, nn, wh1, wss1, wsv1, bs1, wv1,
             wh2, wss2, wsv2, bs2, wv2, tab0, hout):
        e1 = cc[...] - ca[...]
        e1 = e1 * lax.rsqrt(jnp.sum(e1 * e1, axis=-1, keepdims=True) + EPS)
        u2 = nn[...] - ca[...]
        u2 = u2 - jnp.sum(u2 * e1, axis=-1, keepdims=True) * e1
        e2 = u2 * lax.rsqrt(jnp.sum(u2 * u2, axis=-1, keepdims=True) + EPS)
        ex = [e1[:, k:k + 1] for k in range(3)]
        ey = [e2[:, k:k + 1] for k in range(3)]
        ez = [ex[1] * ey[2] - ex[2] * ey[1],
              ex[2] * ey[0] - ex[0] * ey[2],
              ex[0] * ey[1] - ex[1] * ey[0]]
        rmat = {}
        cols = [ca[...]]
        for a in range(3):
            for bb, ee in enumerate((ex, ey, ez)):
                rab = _nan_to_num(ee[a])
                rmat[(a, bb)] = rab
                cols.append(rab)
        zero1 = jnp.zeros_like(ca[:, :1])
        tab0[...] = jnp.concatenate(cols + [zero1] * 4, axis=-1)

        v_c = [nv[:, c * 3:(c + 1) * 3] for c in range(3)]
        vr = [rmat[(0, i)] * v_c[0] + rmat[(1, i)] * v_c[1]
              + rmat[(2, i)] * v_c[2] for i in range(3)]
        vh_c = [sum(vc[:, k:k + 1] * wh1[k:k + 1, :] for k in range(3))
                for vc in vr]
        s = _mm(ns[...], wss1[...]) + _mm(_vnorm(vh_c), wsv1[...]) + bs1[...]
        v = [_mm(vh, wv1[...]) for vh in vh_c]
        s, v = _ln_blk(s, v, 32.0)
        vh_c = [_mm(vc, wh2[...]) for vc in v]
        s = _mm(s, wss2[...]) + _mm(_vnorm(vh_c), wsv2[...]) + bs2[...]
        v = [_mm(vh, wv2[...]) for vh in vh_c]
        vo = [rmat[(i, 0)] * v[0] + rmat[(i, 1)] * v[1]
              + rmat[(i, 2)] * v[2] for i in range(3)]
        hout[...] = jnp.concatenate([s] + vo, axis=-1)

    ws = [w1['wh'], w1['ws'][:6], w1['ws'][6:], w1['bs'][None, :], w1['wv'],
          w2['wh'], w2['ws'][:128], w2['ws'][128:], w2['bs'][None, :],
          w2['wv']]
    return pl.pallas_call(
        body, grid=(N_NODES // NB,),
        in_specs=[_row_spec(NB, 6), _row_spec(NB, 9), _row_spec(NB, 3),
                  _row_spec(NB, 3), _row_spec(NB, 3)]
        + [_w_spec(w.shape) for w in ws],
        out_specs=[_row_spec(NB, 16), _row_spec(NB, 224)],
        out_shape=[jax.ShapeDtypeStruct((N_NODES, 16), jnp.float32),
                   jax.ShapeDtypeStruct((N_NODES, 224), jnp.float32)],
    )(node_s, node_v9, pos_CA, pos_C, pos_N, *ws)


def _edge_embed(pi, pj, dij, w1, w2):
    freq = np.exp(np.arange(0, NUM_POS, 2, dtype=np.float32)
                  * (-np.log(10000.0) / NUM_POS))[None, :]
    mu = np.linspace(0.0, 20.0, NUM_RBF).astype(np.float32)[None, :]
    sigma = 20.0 / NUM_RBF

    def body(pi_r, pj_r, d_r, wh1, wsr1, wspc1, wsps1, wsv1, bs1, wv1,
             wh2, wss2, wsv2, bs2, wv2, es_o, ev_o):
        evec = pi_r[:, 0:3] - pj_r[:, 0:3]
        d2 = jnp.sum(evec * evec, axis=-1, keepdims=True)
        dist = jnp.sqrt(d2 + EPS)
        rbf = jnp.exp(-(((dist - mu) / sigma) ** 2))
        ang = d_r[...] * freq
        u = evec * lax.rsqrt(d2 + EPS)
        u_c = [u[:, k:k + 1] for k in range(3)]
        rc = pi_r[...]
        ur = _rot_in_blk(rc, u_c)
        vh_c = [uc * wh1[0:1, :] for uc in ur]
        s = (_mm(rbf, wsr1[...]) + _mm(jnp.cos(ang), wspc1[...])
             + _mm(jnp.sin(ang), wsps1[...])
             + _mm(_vnorm(vh_c), wsv1[...]) + bs1[...])
        v = [_mm(vh, wv1[...]) for vh in vh_c]
        s, v = _ln_blk(s, v, 16.0)
        vh_c = [_mm(vc, wh2[...]) for vc in v]
        s = _mm(s, wss2[...]) + _mm(_vnorm(vh_c), wsv2[...]) + bs2[...]
        v = [_mm(vh, wv2[...]) for vh in vh_c]
        vo = _rot_out_blk(rc, v)
        es_o[...] = s
        ev_o[...] = jnp.concatenate(vo, axis=-1)

    ws = [w1['wh'], w1['ws'][:16], w1['ws'][16:24], w1['ws'][24:32],
          w1['ws'][32:], w1['bs'][None, :], w1['wv'],
          w2['wh'], w2['ws'][:64], w2['ws'][64:], w2['bs'][None, :],
          w2['wv']]
    return pl.pallas_call(
        body, grid=(EP // EB,),
        in_specs=[_row_spec(EB, 16), _row_spec(EB, 16), _row_spec(EB, 1)]
        + [_w_spec(w.shape) for w in ws],
        out_specs=[_row_spec(EB, 64), _row_spec(EB, 48)],
        out_shape=[jax.ShapeDtypeStruct((EP, 64), jnp.float32),
                   jax.ShapeDtypeStruct((EP, 48), jnp.float32)],
    )(pi, pj, dij, *ws)


def _messages(dsrc, ssrc, es, ev, w1, w2, dec):
    td = dsrc.shape[1]

    def body(d_r, s_r, es_r, ev_r, whd, whs, whe, wsd, wss, wse, wsq,
             wsv, bs1, wv1, wh2, wss2, wsv2, bs2, wv2, ms_o, mv_o):
        ds = d_r[:, 0:128]
        ss = s_r[:, 0:128]
        dv = _vcols(d_r[...], 128, 32)
        sv = _vcols(s_r[...], 128, 32)
        evc = _vcols(ev_r[...], 0, 16)
        vh_c = [_mm(dv[c], whd[...]) + _mm(sv[c], whs[...])
                + _mm(evc[c], whe[...]) for c in range(3)]
        s = (_mm(ds, wsd[...]) + _mm(ss, wss[...]) + _mm(es_r[...], wse[...])
             + _mm(_vnorm(vh_c), wsv[...]) + bs1[...])
        if dec:
            s = s + _mm(s_r[:, 224:244], wsq[...])
        s = jnp.maximum(s, 0.0)
        v = [_mm(vh, wv1[...]) for vh in vh_c]
        g = jax.nn.sigmoid(_vnorm(v))
        v = [vc * g for vc in v]
        vh_c = [_mm(vc, wh2[...]) for vc in v]
        s = _mm(s, wss2[...]) + _mm(_vnorm(vh_c), wsv2[...]) + bs2[...]
        v = [_mm(vh, wv2[...]) for vh in vh_c]
        ms_o[...] = s
        one = jnp.ones_like(s[:, :1])
        mv_o[...] = jnp.concatenate(
            v + [one] + [jnp.zeros_like(one)] * 15, axis=-1)

    if dec:
        wsq = w1['ws'][320:340]
        wsvn = w1['ws'][340:]
    else:
        wsq = w1['ws'][:1]
        wsvn = w1['ws'][320:]
    ws = [w1['wh'][:32], w1['wh'][32:64], w1['wh'][64:],
          w1['ws'][:128], w1['ws'][128:256], w1['ws'][256:320], wsq, wsvn,
          w1['bs'][None, :], w1['wv'],
          w2['wh'], w2['ws'][:128], w2['ws'][128:], w2['bs'][None, :],
          w2['wv']]
    return pl.pallas_call(
        body, grid=(EP // EB,),
        in_specs=[_row_spec(EB, td), _row_spec(EB, td), _row_spec(EB, 64),
                  _row_spec(EB, 48)] + [_w_spec(w.shape) for w in ws],
        out_specs=[_row_spec(EB, 128), _row_spec(EB, 112)],
        out_shape=[jax.ShapeDtypeStruct((EP, 128), jnp.float32),
                   jax.ShapeDtypeStruct((EP, 112), jnp.float32)],
    )(dsrc, ssrc, es, ev, *ws)


def _node_update(h, tab0, msp, mvp, w1, w2):
    def body(h_r, r_r, msa, msb, mva, mvb, wh1, wss1, wsv1, bs1, wv1,
             wh2, wss2, wsv2, bs2, wv2, h_o):
        cnt = mva[:, 96:97] + mvb[:, 96:97]
        inv = 1.0 / jnp.maximum(cnt, 1.0)
        s = h_r[:, 0:128] + (msa[...] + msb[...]) * inv
        v = [h_r[:, 128 + c * 32:160 + c * 32]
             + (mva[:, c * 32:(c + 1) * 32] + mvb[:, c * 32:(c + 1) * 32])
             * inv for c in range(3)]
        s, v = _ln_blk(s, v, 32.0)
        rc = r_r[...]
        vr = _rot_in_blk(rc, v)
        vh_c = [_mm(vc, wh1[...]) for vc in vr]
        fs = jnp.maximum(
            _mm(s, wss1[...]) + _mm(_vnorm(vh_c), wsv1[...]) + bs1[...], 0.0)
        fv = [_mm(vh, wv1[...]) for vh in vh_c]
        g = jax.nn.sigmoid(_vnorm(fv))
        fv = [vc * g for vc in fv]
        vh_c = [_mm(vc, wh2[...]) for vc in fv]
        fs = _mm(fs, wss2[...]) + _mm(_vnorm(vh_c), wsv2[...]) + bs2[...]
        fv = [_mm(vh, wv2[...]) for vh in vh_c]
        fv = _rot_out_blk(rc, fv)
        s2, v2 = _ln_blk(s + fs, [v[c] + fv[c] for c in range(3)], 32.0)
        h_o[...] = jnp.concatenate([s2] + v2, axis=-1)

    msa, msb = msp[:N_NODES], msp[N_NODES:]
    mva, mvb = mvp[:N_NODES], mvp[N_NODES:]
    ws = [w1['wh'], w1['ws'][:128], w1['ws'][128:], w1['bs'][None, :],
          w1['wv'],
          w2['wh'], w2['ws'][:128], w2['ws'][128:], w2['bs'][None, :],
          w2['wv']]
    return pl.pallas_call(
        body, grid=(N_NODES // NB,),
        in_specs=[_row_spec(NB, 224), _row_spec(NB, 16), _row_spec(NB, 128),
                  _row_spec(NB, 128), _row_spec(NB, 112), _row_spec(NB, 112)]
        + [_w_spec(w.shape) for w in ws],
        out_specs=_row_spec(NB, 224),
        out_shape=jax.ShapeDtypeStruct((N_NODES, 224), jnp.float32),
    )(h, tab0, msa, msb, mva, mvb, *ws)


def _head(h, tab0, wo, dense):
    def body(h_r, r_r, wh, wss, wsv, bs, w1, b1, w2, b2, out):
        s = h_r[:, 0:128]
        v = _rot_in_blk(r_r[...], _vcols(h_r[...], 128, 32))
        s, v = _ln_blk(s, v, 32.0)
        vh_c = [_mm(vc, wh[...]) for vc in v]
        o = _mm(s, wss[...]) + _mm(_vnorm(vh_c), wsv[...]) + bs[...]
        hh = jnp.maximum(_mm(o, w1[...]) + b1[...], 0.0)
        out[...] = _mm(hh, w2[...]) + b2[...]

    ws = [wo['wh'], wo['ws'][:128], wo['ws'][128:], wo['bs'][None, :],
          dense['w1'], dense['b1'][None, :], dense['w2'],
          dense['b2'][None, :]]
    return pl.pallas_call(
        body, grid=(N_NODES // NB,),
        in_specs=[_row_spec(NB, 224), _row_spec(NB, 16)]
        + [_w_spec(w.shape) for w in ws],
        out_specs=_row_spec(NB, 20),
        out_shape=jax.ShapeDtypeStruct((N_NODES, 20), jnp.float32),
    )(h, tab0, *ws)


def _pad_idx(x, fill):
    return jnp.concatenate(
        [x, jnp.full((EP - N_EDGES,), fill, jnp.int32)]).reshape(NBLK, IBLK)


def kernel(node_s, node_v, pos_CA, pos_C, pos_N, seq, edge_index, params):
    i = edge_index[0].astype(jnp.int32)
    j = edge_index[1].astype(jnp.int32)
    fwd = j < i

    idx_ij = jnp.concatenate([_pad_idx(i, 0), _pad_idx(j, 0)], axis=0)
    idx_scat = _pad_idx(i, TRASH)
    idx_dij = jnp.concatenate(
        [_pad_idx(jnp.where(fwd, i, i + N_NODES), 0),
         _pad_idx(jnp.where(fwd, j, j + N_NODES), 0)], axis=0)
    dij = jnp.concatenate(
        [(i - j).astype(jnp.float32),
         jnp.zeros((EP - N_EDGES,), jnp.float32)]).reshape(EP, 1)

    node_v9 = jnp.transpose(node_v, (0, 2, 1)).reshape(N_NODES, 9)
    tab0, h = _node_init(node_s, node_v9, pos_CA, pos_C, pos_N,
                         params['W_node'][0], params['W_node'][1])

    pij = _sc_gather(tab0, idx_ij)
    es, ev = _edge_embed(pij[:EP], pij[EP:], dij,
                         params['W_edge'][0], params['W_edge'][1])

    for lp in params['enc']:
        g = _sc_gather(h, idx_ij)
        ms, mv = _messages(g[:EP], g[EP:], es, ev,
                           lp['msg'][0], lp['msg'][1], dec=False)
        msp = _sc_scatter_add(ms, idx_scat)
        mvp = _sc_scatter_add(mv, idx_scat)
        h = _node_update(h, tab0, msp, mvp, lp['ff'][0], lp['ff'][1])

    aa = params['aa_embed'][seq]
    enc_half = jnp.concatenate(
        [h, jnp.zeros((N_NODES, 32), jnp.float32)], axis=-1)

    for lp in params['dec']:
        dec_tab = jnp.concatenate([
            jnp.concatenate([h, aa, jnp.zeros((N_NODES, 12), jnp.float32)],
                            axis=-1),
            enc_half], axis=0)
        g = _sc_gather(dec_tab, idx_dij)
        ms, mv = _messages(g[:EP], g[EP:], es, ev,
                           lp['msg'][0], lp['msg'][1], dec=True)
        msp = _sc_scatter_add(ms, idx_scat)
        mvp = _sc_scatter_add(mv, idx_scat)
        h = _node_update(h, tab0, msp, mvp, lp['ff'][0], lp['ff'][1])

    return _head(h, tab0, params['W_out'], params['dense'])


# speedup vs baseline: 6.3715x; 6.3715x over previous
import functools

import jax
import jax.numpy as jnp
import numpy as np
from jax import lax
from jax.experimental import pallas as pl
from jax.experimental.pallas import tpu as pltpu
from jax.experimental.pallas import tpu_sc as plsc

N_NODES = 10000
N_EDGES = N_NODES * 16
NUM_RBF = 16
NUM_POS = 16
EPS = 1e-8

IBLK = 128
EP = 163840
NBLK = EP // IBLK
EB = 1024
NB = 1000
TRASH = N_NODES
NHALF = N_NODES // 2
NPH = 6000

_NC = 2
_NS = 16
_NW = _NC * _NS


def _sc_gather(table, idx2d):
    nblk = idx2d.shape[0]
    d = table.shape[1]
    per_w = nblk // _NW
    mesh = plsc.VectorSubcoreMesh(core_axis_name="c", subcore_axis_name="s")

    @functools.partial(
        pl.kernel, mesh=mesh,
        out_type=jax.ShapeDtypeStruct((nblk * IBLK, d), jnp.float32),
        scratch_types=[
            pltpu.VMEM((IBLK,), jnp.int32),
            pltpu.VMEM((IBLK, d), jnp.float32),
            pltpu.SemaphoreType.DMA,
        ],
        compiler_params=pltpu.CompilerParams(use_tc_tiling_on_sc=False),
    )
    def k(table_hbm, idx_hbm, out_hbm, idx_v, rows_v, sem):
        wid = lax.axis_index("s") * _NC + lax.axis_index("c")

        def body(c, carry):
            b = wid * per_w + c
            pltpu.sync_copy(idx_hbm.at[b], idx_v)
            pltpu.async_copy(table_hbm.at[idx_v], rows_v, sem).wait()
            pltpu.sync_copy(rows_v, out_hbm.at[pl.ds(b * IBLK, IBLK)])
            return carry

        lax.fori_loop(0, per_w, body, 0)

    return k(table, idx2d)


def _sc_scatter_add(vals, idx2d):
    d = vals.shape[1]
    per_t = NBLK // _NS
    zrows = NPH // _NS
    mesh = plsc.VectorSubcoreMesh(core_axis_name="c", subcore_axis_name="s")

    @functools.partial(
        pl.kernel, mesh=mesh,
        out_type=jax.ShapeDtypeStruct((2 * NPH, d), jnp.float32),
        scratch_types=[
            pltpu.VMEM((IBLK,), jnp.int32),
            pltpu.VMEM((IBLK, d), jnp.float32),
            pltpu.VMEM((zrows, d), jnp.float32),
            pltpu.VMEM_SHARED((NPH, d), jnp.float32),
        ],
        compiler_params=pltpu.CompilerParams(use_tc_tiling_on_sc=False),
    )
    def k(vals_hbm, idx_hbm, out_hbm, idx_v, vbuf, zbuf, acc):
        cid = lax.axis_index("c")
        sid = lax.axis_index("s")
        lo = cid * NHALF

        def zrow(r, carry):
            for q in range(d // 16):
                zbuf[r, pl.ds(q * 16, 16)] = jnp.zeros((16,), jnp.float32)
            return carry

        lax.fori_loop(0, zrows, zrow, 0)
        pltpu.sync_copy(zbuf, acc.at[pl.ds(sid * zrows, zrows)])
        plsc.subcore_barrier()

        def body(c, carry):
            b = sid * per_t + c
            pltpu.sync_copy(idx_hbm.at[b], idx_v)
            pltpu.sync_copy(vals_hbm.at[pl.ds(b * IBLK, IBLK)], vbuf)
            for q in range(IBLK // 16):
                v = idx_v[pl.ds(q * 16, 16)] - lo
                ok = (v >= 0) & (v < NHALF)
                idx_v[pl.ds(q * 16, 16)] = jnp.where(ok, v, NHALF)
            pltpu.sync_copy(vbuf, acc.at[idx_v], add=True)
            return carry

        lax.fori_loop(0, per_t, body, 0)
        plsc.subcore_barrier()

        pltpu.sync_copy(acc.at[pl.ds(sid * zrows, zrows)],
                        zbuf.at[pl.ds(0, zrows)])
        pltpu.sync_copy(zbuf.at[pl.ds(0, zrows)],
                        out_hbm.at[pl.ds(cid * NPH + sid * zrows, zrows)])

    return k(vals, idx2d)


def _mm(a, b):
    return jnp.dot(a, b, preferred_element_type=jnp.float32)


def _rot_in_blk(rc, v_c):
    def r(a, b):
        return rc[:, 3 + 3 * a + b:4 + 3 * a + b]
    return [r(0, i) * v_c[0] + r(1, i) * v_c[1] + r(2, i) * v_c[2]
            for i in range(3)]


def _rot_out_blk(rc, v_c):
    def r(a, b):
        return rc[:, 3 + 3 * a + b:4 + 3 * a + b]
    return [r(i, 0) * v_c[0] + r(i, 1) * v_c[1] + r(i, 2) * v_c[2]
            for i in range(3)]


def _ln_blk(s, v_c, nvec):
    mu = jnp.mean(s, axis=-1, keepdims=True)
    var = jnp.mean((s - mu) * (s - mu), axis=-1, keepdims=True)
    s = (s - mu) * lax.rsqrt(var + 1e-5)
    sq = v_c[0] * v_c[0] + v_c[1] * v_c[1] + v_c[2] * v_c[2]
    rms = lax.rsqrt(jnp.sum(sq, axis=-1, keepdims=True) / nvec + EPS)
    return s, [vc * rms for vc in v_c]


def _vnorm(vh_c):
    return jnp.sqrt(vh_c[0] * vh_c[0] + vh_c[1] * vh_c[1]
                    + vh_c[2] * vh_c[2] + EPS)


def _nan_to_num(x):
    x = jnp.where(x != x, 0.0, x)
    return jnp.clip(x, -3.4028235e38, 3.4028235e38)


def _vcols(h, base, nv):
    return [h[:, base + c * nv: base + (c + 1) * nv] for c in range(3)]


def _row_spec(b, d):
    return pl.BlockSpec((b, d), lambda g: (g, 0))


def _w_spec(shape):
    nd = len(shape)
    return pl.BlockSpec(shape, lambda g: (0,) * nd)


def _node_init(node_s, node_v9, pos_CA, pos_C, pos_N, w1, w2):
    def body(ns, nv, ca, cc, nn, wh1, wss1, wsv1, bs1, wv1,
             wh2, wss2, wsv2, bs2, wv2, tab0, hout):
        e1 = cc[...] - ca[...]
        e1 = e1 * lax.rsqrt(jnp.sum(e1 * e1, axis=-1, keepdims=True) + EPS)
        u2 = nn[...] - ca[...]
        u2 = u2 - jnp.sum(u2 * e1, axis=-1, keepdims=True) * e1
        e2 = u2 * lax.rsqrt(jnp.sum(u2 * u2, axis=-1, keepdims=True) + EPS)
        ex = [e1[:, k:k + 1] for k in range(3)]
        ey = [e2[:, k:k + 1] for k in range(3)]
        ez = [ex[1] * ey[2] - ex[2] * ey[1],
              ex[2] * ey[0] - ex[0] * ey[2],
              ex[0] * ey[1] - ex[1] * ey[0]]
        rmat = {}
        cols = [ca[...]]
        for a in range(3):
            for bb, ee in enumerate((ex, ey, ez)):
                rab = _nan_to_num(ee[a])
                rmat[(a, bb)] = rab
                cols.append(rab)
        zero1 = jnp.zeros_like(ca[:, :1])
        tab0[...] = jnp.concatenate(cols + [zero1] * 4, axis=-1)

        v_c = [nv[:, c * 3:(c + 1) * 3] for c in range(3)]
        vr = [rmat[(0, i)] * v_c[0] + rmat[(1, i)] * v_c[1]
              + rmat[(2, i)] * v_c[2] for i in range(3)]
        vh_c = [sum(vc[:, k:k + 1] * wh1[k:k + 1, :] for k in range(3))
                for vc in vr]
        s = _mm(ns[...], wss1[...]) + _mm(_vnorm(vh_c), wsv1[...]) + bs1[...]
        v = [_mm(vh, wv1[...]) for vh in vh_c]
        s, v = _ln_blk(s, v, 32.0)
        vh_c = [_mm(vc, wh2[...]) for vc in v]
        s = _mm(s, wss2[...]) + _mm(_vnorm(vh_c), wsv2[...]) + bs2[...]
        v = [_mm(vh, wv2[...]) for vh in vh_c]
        vo = [rmat[(i, 0)] * v[0] + rmat[(i, 1)] * v[1]
              + rmat[(i, 2)] * v[2] for i in range(3)]
        hout[...] = jnp.concatenate([s] + vo, axis=-1)

    ws = [w1['wh'], w1['ws'][:6], w1['ws'][6:], w1['bs'][None, :], w1['wv'],
          w2['wh'], w2['ws'][:128], w2['ws'][128:], w2['bs'][None, :],
          w2['wv']]
    return pl.pallas_call(
        body, grid=(N_NODES // NB,),
        in_specs=[_row_spec(NB, 6), _row_spec(NB, 9), _row_spec(NB, 3),
                  _row_spec(NB, 3), _row_spec(NB, 3)]
        + [_w_spec(w.shape) for w in ws],
        out_specs=[_row_spec(NB, 16), _row_spec(NB, 224)],
        out_shape=[jax.ShapeDtypeStruct((N_NODES, 16), jnp.float32),
                   jax.ShapeDtypeStruct((N_NODES, 224), jnp.float32)],
    )(node_s, node_v9, pos_CA, pos_C, pos_N, *ws)


def _edge_embed(pi, pj, dij, w1, w2):
    sigma = 20.0 / NUM_RBF

    def body(pi_r, pj_r, d_r, wh1, wsr1, wspc1, wsps1, wsv1, bs1, wv1,
             wh2, wss2, wsv2, bs2, wv2, es_o, ev_o):
        mu = lax.broadcasted_iota(jnp.int32, (1, NUM_RBF), 1) \
            .astype(jnp.float32) * (20.0 / (NUM_RBF - 1))
        freq = jnp.exp(
            lax.broadcasted_iota(jnp.int32, (1, NUM_POS // 2), 1)
            .astype(jnp.float32) * (-2.0 * np.log(10000.0) / NUM_POS))
        evec = pi_r[:, 0:3] - pj_r[:, 0:3]
        d2 = jnp.sum(evec * evec, axis=-1, keepdims=True)
        dist = jnp.sqrt(d2 + EPS)
        rbf = jnp.exp(-(((dist - mu) / sigma) ** 2))
        ang = d_r[...] * freq
        u = evec * lax.rsqrt(d2 + EPS)
        u_c = [u[:, k:k + 1] for k in range(3)]
        rc = pi_r[...]
        ur = _rot_in_blk(rc, u_c)
        vh_c = [uc * wh1[0:1, :] for uc in ur]
        s = (_mm(rbf, wsr1[...]) + _mm(jnp.cos(ang), wspc1[...])
             + _mm(jnp.sin(ang), wsps1[...])
             + _mm(_vnorm(vh_c), wsv1[...]) + bs1[...])
        v = [_mm(vh, wv1[...]) for vh in vh_c]
        s, v = _ln_blk(s, v, 16.0)
        vh_c = [_mm(vc, wh2[...]) for vc in v]
        s = _mm(s, wss2[...]) + _mm(_vnorm(vh_c), wsv2[...]) + bs2[...]
        v = [_mm(vh, wv2[...]) for vh in vh_c]
        vo = _rot_out_blk(rc, v)
        es_o[...] = s
        ev_o[...] = jnp.concatenate(vo, axis=-1)

    ws = [w1['wh'], w1['ws'][:16], w1['ws'][16:24], w1['ws'][24:32],
          w1['ws'][32:], w1['bs'][None, :], w1['wv'],
          w2['wh'], w2['ws'][:64], w2['ws'][64:], w2['bs'][None, :],
          w2['wv']]
    return pl.pallas_call(
        body, grid=(EP // EB,),
        in_specs=[_row_spec(EB, 16), _row_spec(EB, 16), _row_spec(EB, 1)]
        + [_w_spec(w.shape) for w in ws],
        out_specs=[_row_spec(EB, 64), _row_spec(EB, 48)],
        out_shape=[jax.ShapeDtypeStruct((EP, 64), jnp.float32),
                   jax.ShapeDtypeStruct((EP, 48), jnp.float32)],
    )(pi, pj, dij, *ws)


def _messages(dsrc, ssrc, es, ev, w1, w2, dec):
    td = dsrc.shape[1]

    def body(d_r, s_r, es_r, ev_r, whd, whs, whe, wsd, wss, wse, wsq,
             wsv, bs1, wv1, wh2, wss2, wsv2, bs2, wv2, ms_o, mv_o):
        ds = d_r[:, 0:128]
        ss = s_r[:, 0:128]
        dv = _vcols(d_r[...], 128, 32)
        sv = _vcols(s_r[...], 128, 32)
        evc = _vcols(ev_r[...], 0, 16)
        vh_c = [_mm(dv[c], whd[...]) + _mm(sv[c], whs[...])
                + _mm(evc[c], whe[...]) for c in range(3)]
        s = (_mm(ds, wsd[...]) + _mm(ss, wss[...]) + _mm(es_r[...], wse[...])
             + _mm(_vnorm(vh_c), wsv[...]) + bs1[...])
        if dec:
            s = s + _mm(s_r[:, 224:244], wsq[...])
        s = jnp.maximum(s, 0.0)
        v = [_mm(vh, wv1[...]) for vh in vh_c]
        g = jax.nn.sigmoid(_vnorm(v))
        v = [vc * g for vc in v]
        vh_c = [_mm(vc, wh2[...]) for vc in v]
        s = _mm(s, wss2[...]) + _mm(_vnorm(vh_c), wsv2[...]) + bs2[...]
        v = [_mm(vh, wv2[...]) for vh in vh_c]
        ms_o[...] = s
        one = jnp.ones_like(s[:, :1])
        mv_o[...] = jnp.concatenate(
            v + [one] + [jnp.zeros_like(one)] * 15, axis=-1)

    if dec:
        wsq = w1['ws'][320:340]
        wsvn = w1['ws'][340:]
    else:
        wsq = w1['ws'][:1]
        wsvn = w1['ws'][320:]
    ws = [w1['wh'][:32], w1['wh'][32:64], w1['wh'][64:],
          w1['ws'][:128], w1['ws'][128:256], w1['ws'][256:320], wsq, wsvn,
          w1['bs'][None, :], w1['wv'],
          w2['wh'], w2['ws'][:128], w2['ws'][128:], w2['bs'][None, :],
          w2['wv']]
    return pl.pallas_call(
        body, grid=(EP // EB,),
        in_specs=[_row_spec(EB, td), _row_spec(EB, td), _row_spec(EB, 64),
                  _row_spec(EB, 48)] + [_w_spec(w.shape) for w in ws],
        out_specs=[_row_spec(EB, 128), _row_spec(EB, 112)],
        out_shape=[jax.ShapeDtypeStruct((EP, 128), jnp.float32),
                   jax.ShapeDtypeStruct((EP, 112), jnp.float32)],
    )(dsrc, ssrc, es, ev, *ws)


def _node_update(h, tab0, msp, mvp, w1, w2):
    def body(h_r, r_r, ms, mv, wh1, wss1, wsv1, bs1, wv1,
             wh2, wss2, wsv2, bs2, wv2, h_o):
        inv = 1.0 / jnp.maximum(mv[:, 96:97], 1.0)
        s = h_r[:, 0:128] + ms[...] * inv
        v = [h_r[:, 128 + c * 32:160 + c * 32]
             + mv[:, c * 32:(c + 1) * 32] * inv for c in range(3)]
        s, v = _ln_blk(s, v, 32.0)
        rc = r_r[...]
        vr = _rot_in_blk(rc, v)
        vh_c = [_mm(vc, wh1[...]) for vc in vr]
        fs = jnp.maximum(
            _mm(s, wss1[...]) + _mm(_vnorm(vh_c), wsv1[...]) + bs1[...], 0.0)
        fv = [_mm(vh, wv1[...]) for vh in vh_c]
        g = jax.nn.sigmoid(_vnorm(fv))
        fv = [vc * g for vc in fv]
        vh_c = [_mm(vc, wh2[...]) for vc in fv]
        fs = _mm(fs, wss2[...]) + _mm(_vnorm(vh_c), wsv2[...]) + bs2[...]
        fv = [_mm(vh, wv2[...]) for vh in vh_c]
        fv = _rot_out_blk(rc, fv)
        s2, v2 = _ln_blk(s + fs, [v[c] + fv[c] for c in range(3)], 32.0)
        h_o[...] = jnp.concatenate([s2] + v2, axis=-1)

    ws = [w1['wh'], w1['ws'][:128], w1['ws'][128:], w1['bs'][None, :],
          w1['wv'],
          w2['wh'], w2['ws'][:128], w2['ws'][128:], w2['bs'][None, :],
          w2['wv']]
    nh_blocks = NHALF // NB

    def seg_map(g):
        return (jnp.where(g < nh_blocks, g, g + (NPH - NHALF) // NB), 0)

    seg_spec_ms = pl.BlockSpec((NB, 128), seg_map)
    seg_spec_mv = pl.BlockSpec((NB, 112), seg_map)
    return pl.pallas_call(
        body, grid=(N_NODES // NB,),
        in_specs=[_row_spec(NB, 224), _row_spec(NB, 16), seg_spec_ms,
                  seg_spec_mv]
        + [_w_spec(w.shape) for w in ws],
        out_specs=_row_spec(NB, 224),
        out_shape=jax.ShapeDtypeStruct((N_NODES, 224), jnp.float32),
    )(h, tab0, msp, mvp, *ws)


def _head(h, tab0, wo, dense):
    def body(h_r, r_r, wh, wss, wsv, bs, w1, b1, w2, b2, out):
        s = h_r[:, 0:128]
        v = _rot_in_blk(r_r[...], _vcols(h_r[...], 128, 32))
        s, v = _ln_blk(s, v, 32.0)
        vh_c = [_mm(vc, wh[...]) for vc in v]
        o = _mm(s, wss[...]) + _mm(_vnorm(vh_c), wsv[...]) + bs[...]
        hh = jnp.maximum(_mm(o, w1[...]) + b1[...], 0.0)
        out[...] = _mm(hh, w2[...]) + b2[...]

    ws = [wo['wh'], wo['ws'][:128], wo['ws'][128:], wo['bs'][None, :],
          dense['w1'], dense['b1'][None, :], dense['w2'],
          dense['b2'][None, :]]
    return pl.pallas_call(
        body, grid=(N_NODES // NB,),
        in_specs=[_row_spec(NB, 224), _row_spec(NB, 16)]
        + [_w_spec(w.shape) for w in ws],
        out_specs=_row_spec(NB, 20),
        out_shape=jax.ShapeDtypeStruct((N_NODES, 20), jnp.float32),
    )(h, tab0, *ws)


def _pad_idx(x, fill):
    return jnp.concatenate(
        [x, jnp.full((EP - N_EDGES,), fill, jnp.int32)]).reshape(NBLK, IBLK)


def kernel(node_s, node_v, pos_CA, pos_C, pos_N, seq, edge_index, params):
    i = edge_index[0].astype(jnp.int32)
    j = edge_index[1].astype(jnp.int32)
    fwd = j < i

    idx_ij = jnp.concatenate([_pad_idx(i, 0), _pad_idx(j, 0)], axis=0)
    idx_scat = _pad_idx(i, TRASH)
    idx_dij = jnp.concatenate(
        [_pad_idx(jnp.where(fwd, i, i + N_NODES), 0),
         _pad_idx(jnp.where(fwd, j, j + N_NODES), 0)], axis=0)
    dij = jnp.concatenate(
        [(i - j).astype(jnp.float32),
         jnp.zeros((EP - N_EDGES,), jnp.float32)]).reshape(EP, 1)

    node_v9 = jnp.transpose(node_v, (0, 2, 1)).reshape(N_NODES, 9)
    tab0, h = _node_init(node_s, node_v9, pos_CA, pos_C, pos_N,
                         params['W_node'][0], params['W_node'][1])

    pij = _sc_gather(tab0, idx_ij)
    es, ev = _edge_embed(pij[:EP], pij[EP:], dij,
                         params['W_edge'][0], params['W_edge'][1])

    for lp in params['enc']:
        g = _sc_gather(h, idx_ij)
        ms, mv = _messages(g[:EP], g[EP:], es, ev,
                           lp['msg'][0], lp['msg'][1], dec=False)
        msp = _sc_scatter_add(ms, idx_scat)
        mvp = _sc_scatter_add(mv, idx_scat)
        h = _node_update(h, tab0, msp, mvp, lp['ff'][0], lp['ff'][1])

    aa = params['aa_embed'][seq]
    enc_half = jnp.concatenate(
        [h, jnp.zeros((N_NODES, 32), jnp.float32)], axis=-1)

    for lp in params['dec']:
        dec_tab = jnp.concatenate([
            jnp.concatenate([h, aa, jnp.zeros((N_NODES, 12), jnp.float32)],
                            axis=-1),
            enc_half], axis=0)
        g = _sc_gather(dec_tab, idx_dij)
        ms, mv = _messages(g[:EP], g[EP:], es, ev,
                           lp['msg'][0], lp['msg'][1], dec=True)
        msp = _sc_scatter_add(ms, idx_scat)
        mvp = _sc_scatter_add(mv, idx_scat)
        h = _node_update(h, tab0, msp, mvp, lp['ff'][0], lp['ff'][1])

    return _head(h, tab0, params['W_out'], params['dense'])
